# baseline (speedup 1.0000x reference)
import jax
import jax.numpy as jnp
from jax.experimental import pallas as pl
from jax.experimental.pallas import tpu as pltpu

_B, _S, _D = 4, 8192, 4096
_R = 2048
_IDX = 3


def _select_body(in_ref, out_ref):
    out_ref[...] = in_ref[0, :, _IDX].reshape(_R // 128, 128)


@jax.jit
def kernel(input_):
    compact = pl.pallas_call(
        _select_body,
        grid=(_B, _S // _R),
        in_specs=[
            pl.BlockSpec((1, _R, 128), lambda b, i: (b, i, 0)),
        ],
        out_specs=pl.BlockSpec(
            (_R // 128, 128), lambda b, i: (b * (_S // _R) + i, 0)
        ),
        out_shape=jax.ShapeDtypeStruct((_B * _S // 128, 128), jnp.float32),
        compiler_params=pltpu.CompilerParams(
            dimension_semantics=("arbitrary", "arbitrary"),
        ),
    )(input_)
    return compact.reshape(_B, _S, 1)


# speedup vs baseline: 1.3518x; 1.3518x over previous
import jax
import jax.numpy as jnp
from jax.experimental import pallas as pl
from jax.experimental.pallas import tpu as pltpu

_B, _S, _D = 4, 8192, 4096
_R = 4096
_IDX = 3


def _select_body(in_ref, out_ref):
    out_ref[...] = in_ref[0, :, _IDX].reshape(_R // 128, 128)


@jax.jit
def kernel(input_):
    compact = pl.pallas_call(
        _select_body,
        grid=(_B, _S // _R),
        in_specs=[
            pl.BlockSpec((1, _R, 128), lambda b, i: (b, i, 0)),
        ],
        out_specs=pl.BlockSpec(
            (_R // 128, 128), lambda b, i: (b * (_S // _R) + i, 0)
        ),
        out_shape=jax.ShapeDtypeStruct((_B * _S // 128, 128), jnp.float32),
        compiler_params=pltpu.CompilerParams(
            dimension_semantics=("arbitrary", "arbitrary"),
        ),
    )(input_)
    return compact.reshape(_B, _S, 1)


# speedup vs baseline: 1.6189x; 1.1976x over previous
import jax
import jax.numpy as jnp
from jax.experimental import pallas as pl
from jax.experimental.pallas import tpu as pltpu

_B, _S, _D = 4, 8192, 4096
_R = 8192
_IDX = 3


def _select_body(in_ref, out_ref):
    out_ref[...] = in_ref[0, :, _IDX].reshape(_R // 128, 128)


@jax.jit
def kernel(input_):
    compact = pl.pallas_call(
        _select_body,
        grid=(_B, _S // _R),
        in_specs=[
            pl.BlockSpec((1, _R, 128), lambda b, i: (b, i, 0)),
        ],
        out_specs=pl.BlockSpec(
            (_R // 128, 128), lambda b, i: (b * (_S // _R) + i, 0)
        ),
        out_shape=jax.ShapeDtypeStruct((_B * _S // 128, 128), jnp.float32),
        compiler_params=pltpu.CompilerParams(
            dimension_semantics=("arbitrary", "arbitrary"),
        ),
    )(input_)
    return compact.reshape(_B, _S, 1)


# speedup vs baseline: 1.6988x; 1.0493x over previous
import jax
import jax.numpy as jnp
from jax.experimental import pallas as pl
from jax.experimental.pallas import tpu as pltpu

_B, _S, _D = 4, 8192, 4096
_CH = 2048
_NQ = 4
_IDX = 3
_NCHUNK = _B * _S // _CH


def _select_body(in_hbm, out_ref, bufs, sems):
    chunks = [(b, i) for b in range(_B) for i in range(_S // _CH)]
    copies = [
        pltpu.make_async_copy(
            in_hbm.at[b, pl.ds(i * _CH, _CH), pl.ds(0, 128)],
            bufs.at[k % _NQ],
            sems.at[k % _NQ],
        )
        for k, (b, i) in enumerate(chunks)
    ]
    for k in range(_NQ):
        copies[k].start()
    for k in range(_NCHUNK):
        copies[k].wait()
        vals = bufs[k % _NQ, :, _IDX]
        out_ref[pl.ds(k * (_CH // 128), _CH // 128), :] = vals.reshape(
            _CH // 128, 128
        )
        if k + _NQ < _NCHUNK:
            copies[k + _NQ].start()


@jax.jit
def kernel(input_):
    compact = pl.pallas_call(
        _select_body,
        in_specs=[pl.BlockSpec(memory_space=pl.ANY)],
        out_specs=pl.BlockSpec((_B * _S // 128, 128), lambda: (0, 0)),
        out_shape=jax.ShapeDtypeStruct((_B * _S // 128, 128), jnp.float32),
        scratch_shapes=[
            pltpu.VMEM((_NQ, _CH, 128), jnp.float32),
            pltpu.SemaphoreType.DMA((_NQ,)),
        ],
    )(input_)
    return compact.reshape(_B, _S, 1)
